# baseline (speedup 1.0000x reference)
import dataclasses
import functools

import jax
import jax.numpy as jnp
import numpy as np
from jax import lax
from jax.experimental import pallas as pl
from jax.experimental.pallas import tpu as pltpu
from jax.experimental.pallas import tpu_sc as plsc

_F = 26
_D = 16
_A = 16
_G = 8
_CHUNK = 128


def _sc_gather(emb_table, lin16, eidx, lrow, lsub, batch):
    mesh = plsc.VectorSubcoreMesh(core_axis_name="core", subcore_axis_name="subcore")
    cp = pltpu.CompilerParams()
    for fld, val in (("needs_layout_passes", False),
                     ("use_tc_tiling_on_sc", False)):
        if fld in pltpu.CompilerParams.__dataclass_fields__:
            cp = dataclasses.replace(cp, **{fld: val})

    @functools.partial(
        pl.kernel,
        compiler_params=cp,
        out_type=(
            jax.ShapeDtypeStruct((batch // _G, _F, _G * _D), jnp.float32),
            jax.ShapeDtypeStruct((batch,), jnp.float32),
        ),
        mesh=mesh,
        scratch_types=[
            pltpu.VMEM((_CHUNK,), jnp.int32),
            pltpu.VMEM((_CHUNK, _D), jnp.float32),
            pltpu.VMEM((_CHUNK // _G, 1, _G * _D), jnp.float32),
            pltpu.VMEM((_CHUNK,), jnp.int32),
            pltpu.VMEM((_CHUNK,), jnp.int32),
            pltpu.VMEM((_CHUNK, _D), jnp.float32),
            pltpu.VMEM((_CHUNK,), jnp.float32),
            pltpu.SemaphoreType.DMA,
            pltpu.SemaphoreType.DMA,
        ],
    )
    def gather_kernel(emb_hbm, lin_hbm, ei_hbm, lr_hbm, ls_hbm, oe_hbm, ol_hbm,
                      eidx_v, erows_v, pack_v, lrow_v, lsub_v, lrows_v, acc_v,
                      sem_e, sem_l):
        wid = lax.axis_index("subcore") * 2 + lax.axis_index("core")
        b0 = wid * _CHUNK
        zeros16 = jnp.zeros((16,), jnp.float32)

        @pl.loop(0, _CHUNK // 16)
        def _(c):
            acc_v[pl.ds(c * 16, 16)] = zeros16

        @pl.loop(0, _F)
        def _(f):
            base = f * batch + b0
            pltpu.sync_copy(ei_hbm.at[pl.ds(base, _CHUNK)], eidx_v)
            pltpu.async_copy(emb_hbm.at[eidx_v], erows_v, sem_e).wait()

            @pl.loop(0, _CHUNK)
            def _(r):
                pack_v[r // _G, 0, pl.ds((r % _G) * _D, _D)] = erows_v[r, :]

            pltpu.sync_copy(
                pack_v, oe_hbm.at[pl.ds(wid * (_CHUNK // _G), _CHUNK // _G),
                                  pl.ds(f, 1), :])
            pltpu.sync_copy(lr_hbm.at[pl.ds(base, _CHUNK)], lrow_v)
            pltpu.sync_copy(ls_hbm.at[pl.ds(base, _CHUNK)], lsub_v)
            pltpu.async_copy(lin_hbm.at[lrow_v], lrows_v, sem_l).wait()

            @pl.loop(0, _CHUNK // 16)
            def _(c):
                rows = lax.iota(jnp.int32, 16) + c * 16
                cols = lsub_v[pl.ds(c * 16, 16)]
                vals = plsc.load_gather(lrows_v, [rows, cols])
                acc_v[pl.ds(c * 16, 16)] += vals

        pltpu.sync_copy(acc_v, ol_hbm.at[pl.ds(b0, _CHUNK)])

    return gather_kernel(emb_table, lin16, eidx, lrow, lsub)


def _afm_body(e_ref, lin_ref, bdw_ref, bdp_ref, bdf_ref, attn_bt_ref, bias_ref,
              out_ref):
    n_groups = e_ref.shape[0]
    pair_iota = jax.lax.broadcasted_iota(jnp.int32, (_F * _F, 1), 0)
    valid = (pair_iota // _F) < (pair_iota % _F)
    attn_bt = attn_bt_ref[...]
    rows = []
    for g in range(n_groups):
        e8 = e_ref[g, :, :]
        i8 = (e8[:, None, :] * e8[None, :, :]).reshape(_F * _F, _G * _D)
        attn = jnp.maximum(
            jnp.dot(i8, bdw_ref[...], preferred_element_type=jnp.float32)
            + attn_bt, 0.0)
        logits = jnp.dot(attn, bdp_ref[...],
                         preferred_element_type=jnp.float32)
        logits = jnp.where(valid, logits, -1e30)
        mx = jnp.max(logits, axis=0, keepdims=True)
        ex = jnp.where(valid, jnp.exp(logits - mx), 0.0)
        scores = ex / jnp.sum(ex, axis=0, keepdims=True)
        rows.append(jnp.sum(scores * i8, axis=0, keepdims=True))
    ao = jnp.concatenate(rows, axis=0)
    afm = jnp.dot(ao, bdf_ref[...], preferred_element_type=jnp.float32)
    out_ref[...] = afm + lin_ref[...] + bias_ref[...]


def _afm_tc(e_packed, lin_sums8, bdw, bdp, bdf, attn_bt, bias, batch):
    n_groups_blk = _CHUNK // _G
    grid = (batch // _CHUNK,)
    return pl.pallas_call(
        _afm_body,
        grid=grid,
        in_specs=[
            pl.BlockSpec((n_groups_blk, _F, _G * _D), lambda i: (i, 0, 0)),
            pl.BlockSpec((n_groups_blk, _G), lambda i: (i, 0)),
            pl.BlockSpec((_G * _D, _G * _D), lambda i: (0, 0)),
            pl.BlockSpec((_G * _D, _G * _D), lambda i: (0, 0)),
            pl.BlockSpec((_G * _D, _G), lambda i: (0, 0)),
            pl.BlockSpec((1, _G * _D), lambda i: (0, 0)),
            pl.BlockSpec((1, 1), lambda i: (0, 0)),
        ],
        out_specs=pl.BlockSpec((n_groups_blk, _G), lambda i: (i, 0)),
        out_shape=jax.ShapeDtypeStruct((batch // _G, _G), jnp.float32),
    )(e_packed, lin_sums8, bdw, bdp, bdf, attn_bt, bias)


def kernel(x, emb_table, lin_table, lin_bias, attn_W, attn_b, proj_W, proj_b,
           fc_W, fc_b):
    batch, num_fields = x.shape
    field_dim = emb_table.shape[0] // num_fields
    offsets = (jnp.arange(num_fields, dtype=x.dtype) * field_dim)[None, :]
    idxf = (x + offsets).T.reshape(-1)
    lin16 = lin_table.reshape(-1, _D)

    e_packed, lin_sums = _sc_gather(emb_table, lin16, idxf,
                                    idxf // _D, idxf % _D, batch)

    eye = jnp.eye(_G, dtype=jnp.float32)
    bdw = jnp.kron(eye, attn_W)
    bdp = jnp.kron(eye, jnp.outer(proj_W[:, 0], jnp.ones((_D,), jnp.float32)))
    bdf = jnp.kron(eye, fc_W)
    attn_bt = jnp.tile(attn_b, (_G,))[None, :]
    bias = (fc_b + lin_bias).reshape(1, 1)

    out = _afm_tc(e_packed, lin_sums.reshape(batch // _G, _G), bdw, bdp, bdf,
                  attn_bt, bias, batch)
    return out.reshape(batch)
